# baseline (speedup 1.0000x reference)
import jax
import jax.numpy as jnp
from jax.experimental import pallas as pl
from jax.experimental.pallas import tpu as pltpu

B, S, D = 2, 2048, 768
E, K, H = 8, 2, 768
T = B * S
TB = 512
EPAD = 128

_NEG = -1e30


def _moe_kernel(x_ref, gw_ref, gb_ref, w1_ref, b1_ref, w2_ref, b2_ref,
                wo_ref, bo_ref, out_ref):
    xb = x_ref[...]

    logits = jnp.dot(xb, gw_ref[...], preferred_element_type=jnp.float32)
    logits = logits + gb_ref[...]

    lane = jax.lax.broadcasted_iota(jnp.int32, (TB, E), 1)
    big = jnp.int32(E)

    l1 = jnp.max(logits, axis=-1, keepdims=True)
    i1 = jnp.min(jnp.where(logits == l1, lane, big), axis=-1, keepdims=True)
    oh1 = (lane == i1)
    masked = jnp.where(oh1, _NEG, logits)
    l2 = jnp.max(masked, axis=-1, keepdims=True)
    i2 = jnp.min(jnp.where(masked == l2, lane, big), axis=-1, keepdims=True)
    oh2 = (lane == i2)

    r = jnp.exp(l2 - l1)
    w_top1 = 1.0 / (1.0 + r)
    w_top2 = 1.0 - w_top1
    probs = w_top1 * oh1.astype(jnp.float32) + w_top2 * oh2.astype(jnp.float32)

    acc = jnp.zeros((TB, D), dtype=jnp.float32)
    for e in range(E):
        h = jnp.dot(xb, w1_ref[e], preferred_element_type=jnp.float32)
        h = jnp.maximum(h + b1_ref[e][None, :], 0.0)
        y = jnp.dot(h, w2_ref[e], preferred_element_type=jnp.float32)
        y = y + b2_ref[e][None, :]
        acc = acc + y * probs[:, e][:, None]

    out = jnp.dot(acc, wo_ref[...], preferred_element_type=jnp.float32)
    out_ref[...] = out + bo_ref[...]


@jax.jit
def kernel(x, gate_W, gate_b, W1, b1, W2, b2, Wout, bout):
    xf = x.reshape(T, D)

    grid = (T // TB,)
    full = lambda shape: pl.BlockSpec(shape, lambda i: (0,) * len(shape))
    out = pl.pallas_call(
        _moe_kernel,
        grid=grid,
        in_specs=[
            pl.BlockSpec((TB, D), lambda i: (i, 0)),
            full((D, E)),
            full((1, E)),
            full((E, D, H)),
            full((E, H)),
            full((E, H, D)),
            full((E, D)),
            full((D, D)),
            full((1, D)),
        ],
        out_specs=pl.BlockSpec((TB, D), lambda i: (i, 0)),
        out_shape=jax.ShapeDtypeStruct((T, D), jnp.float32),
        compiler_params=pltpu.CompilerParams(
            vmem_limit_bytes=120 * 1024 * 1024,
        ),
    )(xf, gate_W, gate_b.reshape(1, E), W1, b1, W2, b2, Wout,
      bout.reshape(1, D))
    return out.reshape(B, S, D)


# speedup vs baseline: 1.0162x; 1.0162x over previous
import jax
import jax.numpy as jnp
from jax.experimental import pallas as pl
from jax.experimental.pallas import tpu as pltpu

B, S, D = 2, 2048, 768
E, K, H = 8, 2, 768
T = B * S
TB = 1024

_NEG = -1e30


def _moe_kernel(x_ref, gw_ref, gb_ref, w1_ref, b1_ref, w2_ref, b2_ref,
                wo_ref, bo_ref, out_ref):
    xb = x_ref[...]

    logits = jnp.dot(xb, gw_ref[...], preferred_element_type=jnp.float32)
    logits = logits + gb_ref[...]

    lane = jax.lax.broadcasted_iota(jnp.int32, (TB, E), 1)
    big = jnp.int32(E)

    l1 = jnp.max(logits, axis=-1, keepdims=True)
    i1 = jnp.min(jnp.where(logits == l1, lane, big), axis=-1, keepdims=True)
    oh1 = (lane == i1)
    masked = jnp.where(oh1, _NEG, logits)
    l2 = jnp.max(masked, axis=-1, keepdims=True)
    i2 = jnp.min(jnp.where(masked == l2, lane, big), axis=-1, keepdims=True)
    oh2 = (lane == i2)

    r = jnp.exp(l2 - l1)
    w_top1 = 1.0 / (1.0 + r)
    w_top2 = 1.0 - w_top1
    probs = w_top1 * oh1.astype(jnp.float32) + w_top2 * oh2.astype(jnp.float32)

    acc = jnp.zeros((TB, D), dtype=jnp.float32)
    for e in range(E):
        h = jnp.dot(xb, w1_ref[e], preferred_element_type=jnp.float32)
        h = jnp.maximum(h + b1_ref[e][None, :], 0.0)
        y = jnp.dot(h, w2_ref[e], preferred_element_type=jnp.float32)
        y = y + b2_ref[e][None, :]
        acc = acc + y * probs[:, e][:, None]

    out = jnp.dot(acc, wo_ref[...], preferred_element_type=jnp.float32)
    out_ref[...] = out + bo_ref[...]


@jax.jit
def kernel(x, gate_W, gate_b, W1, b1, W2, b2, Wout, bout):
    xf = x.reshape(T, D)

    grid = (T // TB,)
    full = lambda shape: pl.BlockSpec(shape, lambda i: (0,) * len(shape))
    out = pl.pallas_call(
        _moe_kernel,
        grid=grid,
        in_specs=[
            pl.BlockSpec((TB, D), lambda i: (i, 0)),
            full((D, E)),
            full((1, E)),
            full((E, D, H)),
            full((E, H)),
            full((E, H, D)),
            full((E, D)),
            full((D, D)),
            full((1, D)),
        ],
        out_specs=pl.BlockSpec((TB, D), lambda i: (i, 0)),
        out_shape=jax.ShapeDtypeStruct((T, D), jnp.float32),
        compiler_params=pltpu.CompilerParams(
            vmem_limit_bytes=120 * 1024 * 1024,
        ),
    )(xf, gate_W, gate_b.reshape(1, E), W1, b1, W2, b2, Wout,
      bout.reshape(1, D))
    return out.reshape(B, S, D)
